# baseline (speedup 1.0000x reference)
import functools

import jax
import jax.numpy as jnp
from jax import lax
from jax.experimental import pallas as pl
from jax.experimental.pallas import tpu as pltpu
from jax.experimental.pallas import tpu_sc as plsc

CH = 512
RB = 256
CB = 512


def _sc_compact_gather(n_pairs, n_cls, n_ch, b, hw, c, n_units, nw, reps,
                       tab_ref, v_ref, lab_ref, base_ref, qkc_ref, vc_ref,
                       idxo_ref, cnto_ref, lab_v, idx_v, idxs_v, rows_v,
                       vrows_v, cnt_v, gb_v, sb_v, cs_v, sem):
    wid = lax.axis_index("s") * 2 + lax.axis_index("c")
    nchp1 = hw // CH + 1
    hwp = hw + 8
    for rep in range(reps):
        u = wid + rep * nw

        @pl.when(u < n_units)
        def _unit(u=u):
            side = u % 2
            pair = u // 2
            b_idx = pair // n_cls
            cls = pair % n_cls + 1
            t_in = side * b + b_idx
            lab_row = (side * b + b_idx) * n_ch + cls
            pltpu.sync_copy(lab_ref.at[pl.ds(lab_row * hw, hw)], lab_v)
            pltpu.sync_copy(base_ref.at[pl.ds(u * 16, 16)], gb_v)
            pltpu.sync_copy(base_ref.at[pl.ds((n_units + u) * 16, 16)], sb_v)
            lanes = lax.broadcasted_iota(jnp.int32, (16,), 0)
            gbase = gb_v[...]
            sbase = sb_v[...]
            gtrash = gbase + hw
            strash = sbase + hw

            def _init(i, carry):
                idx_v[pl.ds(i * 16, 16)] = gtrash
                idxs_v[pl.ds(i * 16, 16)] = strash
                return carry

            lax.fori_loop(0, (nchp1 * CH) // 16, _init, 0)
            fifteen = jnp.full((16,), 15, jnp.int32)

            def _scan(i, carry):
                off_v, pixg_v, pixs_v = carry
                mi = lab_v[pl.ds(i * 16, 16)]
                m = mi != 0
                csum = plsc.cumsum(mi)
                pos = off_v + csum - mi
                plsc.store_scatter(idx_v, [pos], pixg_v, mask=m)
                plsc.store_scatter(idxs_v, [pos], pixs_v, mask=m)
                cs_v[...] = csum
                total_v = plsc.load_gather(cs_v, [fifteen])
                return off_v + total_v, pixg_v + 16, pixs_v + 16

            off_v, _, _ = lax.fori_loop(
                0, hw // 16, _scan,
                (jnp.zeros((16,), jnp.int32), lanes + gbase, lanes + sbase))
            cnt = jnp.max(off_v)
            cnt_v[...] = off_v
            pltpu.sync_copy(cnt_v, cnto_ref.at[pl.ds(u * 16, 16)])
            pltpu.sync_copy(idxs_v,
                            idxo_ref.at[pl.ds(u * nchp1 * CH, nchp1 * CH)])

            nchunk = (cnt + CH - 1) // CH

            def _gather(j, carry):
                pltpu.async_copy(
                    tab_ref.at[idx_v.at[pl.ds(j * CH, CH)]],
                    rows_v, sem).wait()
                pltpu.sync_copy(rows_v, qkc_ref.at[u].at[pl.ds(j * CH, CH)])
                return carry

            lax.fori_loop(0, nchunk, _gather, 0)

            @pl.when(side == 1)
            def _vgather():
                def _vg(j, carry):
                    pltpu.async_copy(
                        v_ref.at[idx_v.at[pl.ds(j * CH, CH)]],
                        vrows_v, sem).wait()
                    pltpu.sync_copy(vrows_v,
                                    vc_ref.at[pair].at[pl.ds(j * CH, CH)])
                    return carry

                lax.fori_loop(0, nchunk, _vg, 0)


def _sc_scatter(n_pairs, hw, nchp1, colc_ref, idxo_ref, cnto_ref, dense_ref,
                idx2_v, rows8_v, cnt_v, sem):
    wid = lax.axis_index("s") * 2 + lax.axis_index("c")

    @pl.when(wid < n_pairs)
    def _unit():
        ug = 2 * wid
        pltpu.sync_copy(cnto_ref.at[pl.ds(ug * 16, 16)], cnt_v)
        cnt = jnp.max(cnt_v[...])
        pltpu.sync_copy(idxo_ref.at[pl.ds(ug * nchp1 * CH, nchp1 * CH)],
                        idx2_v)
        nchunk = (cnt + CH - 1) // CH

        def _scatter(j, carry):
            pltpu.sync_copy(colc_ref.at[wid].at[pl.ds(j * CH, CH)], rows8_v)
            pltpu.async_copy(rows8_v,
                             dense_ref.at[idx2_v.at[pl.ds(j * CH, CH)]],
                             sem).wait()
            return carry

        lax.fori_loop(0, nchunk, _scatter, 0)


def _sanitize_normalize(src_ref, dst_ref, n, hw, c):
    chn = 512
    nch = hw // chn
    nf = jnp.maximum(n.astype(jnp.float32), 1.0)

    def _pa(i, acc):
        rows = lax.broadcasted_iota(jnp.int32, (chn, 1), 0) + i * chn
        v = jnp.where(rows < n, src_ref[0, pl.ds(i * chn, chn), :], 0.0)
        return acc + jnp.sum(v, axis=0, keepdims=True)

    mean = lax.fori_loop(0, nch, _pa, jnp.zeros((1, c), jnp.float32)) / nf

    def _pb(i, carry):
        rows = lax.broadcasted_iota(jnp.int32, (chn, 1), 0) + i * chn
        v = src_ref[0, pl.ds(i * chn, chn), :]
        bar = jnp.where(rows < n, v - mean, 0.0)
        n2 = jnp.sum(bar * bar, axis=1, keepdims=True)
        dst_ref[pl.ds(i * chn, chn), :] = (
            bar * lax.rsqrt(jnp.where(n2 == 0.0, 1.0, n2))
        ).astype(jnp.bfloat16)
        return carry

    lax.fori_loop(0, nch, _pb, 0)


def _tc_attn(hw, cnt_ref, q_ref, k_ref, v_ref, out_ref, qn_ref, kn_ref,
             w_ref):
    p = pl.program_id(0)
    rb = pl.program_id(1)
    ng = cnt_ref[2 * p * 16]
    nr = cnt_ref[(2 * p + 1) * 16]

    @pl.when(rb == 0)
    def _prep():
        rows_i = lax.broadcasted_iota(jnp.int32, (hw, 1), 0)
        _sanitize_normalize(q_ref, qn_ref, ng, hw, q_ref.shape[2])
        _sanitize_normalize(k_ref, kn_ref, nr, hw, k_ref.shape[2])
        vraw = v_ref[0]
        wcols = jnp.concatenate(
            [vraw[:, 0:3], jnp.ones((hw, 1), jnp.float32),
             jnp.zeros((hw, 4), jnp.float32)], axis=1)
        w_ref[...] = jnp.where(rows_i < nr, wcols, 0.0).astype(jnp.bfloat16)

    @pl.when(rb * RB < ng)
    def _compute():
        q = qn_ref[pl.ds(rb * RB, RB), :]
        ncb = (nr + CB - 1) // CB

        def _col(cb, acc):
            ks = kn_ref[pl.ds(cb * CB, CB), :]
            logits = lax.dot_general(q, ks, (((1,), (1,)), ((), ())),
                                     preferred_element_type=jnp.float32)
            e = jnp.exp(logits).astype(jnp.bfloat16)
            ws = w_ref[pl.ds(cb * CB, CB), :]
            return acc + lax.dot_general(
                e, ws, (((1,), (0,)), ((), ())),
                preferred_element_type=jnp.float32)

        acc = lax.fori_loop(0, ncb, _col, jnp.zeros((RB, 8), jnp.float32))
        res = acc[:, 0:3] / jnp.maximum(acc[:, 3:4], 1e-30)
        out_ref[0] = jnp.concatenate(
            [res, jnp.zeros((RB, 5), jnp.float32)], axis=1)


def _tc_merge(n_cls, n_ch, chm, cnt_ref, col_ref, gl_ref, out_ref):
    ib = pl.program_id(0)
    acc = jnp.full((chm, 3), -1.0, jnp.float32)
    for cidx in range(1, n_ch):
        p = ib * n_cls + cidx - 1
        ng = cnt_ref[2 * p * 16]
        nr = cnt_ref[(2 * p + 1) * 16]
        valid = (ng > 1) & (nr > 1)
        gm = gl_ref[0, :, cidx:cidx + 1] != 0
        sel = valid & gm
        acc = jnp.where(sel, col_ref[cidx - 1, :, 0:3], acc)
    out_ref[0] = acc


def kernel(gray_feature, rgb_feature, rgb_image, gray_label, rgb_label):
    b, c, h, w = gray_feature.shape
    n_ch = gray_label.shape[1]
    hw = h * w
    n_cls = n_ch - 1
    n_pairs = b * n_cls
    n_units = 2 * n_pairs
    hwp = hw + 8
    nchp1 = hw // CH + 1
    n_rb = hw // RB

    gfT = jnp.swapaxes(gray_feature.reshape(b, c, hw), 1, 2)
    rfT = jnp.swapaxes(rgb_feature.reshape(b, c, hw), 1, 2)
    tabcat = jnp.pad(jnp.concatenate([gfT, rfT], axis=0),
                     ((0, 0), (0, 8), (0, 0))).reshape(2 * b * hwp, c)
    vT1 = jnp.pad(jnp.swapaxes(rgb_image.reshape(b, 3, hw), 1, 2),
                  ((0, 0), (0, 8), (0, 5)))
    vT = jnp.concatenate([vT1, vT1], axis=0).reshape(2 * b * hwp, 8)
    labcat = jnp.stack([gray_label.reshape(b, n_ch, hw),
                        rgb_label.reshape(b, n_ch, hw)],
                       axis=0).reshape(2 * b * n_ch * hw)
    glT = jnp.swapaxes(gray_label.reshape(b, n_ch, hw), 1, 2)
    rlT = jnp.swapaxes(rgb_label.reshape(b, n_ch, hw), 1, 2)

    uu = jnp.arange(n_units, dtype=jnp.int32)
    t_in_u = (uu % 2) * b + (uu // 2) // n_cls
    gbases = jnp.repeat(t_in_u * hwp, 16)
    sbases = jnp.repeat((uu // 2) * hwp, 16)
    bases = jnp.concatenate([gbases, sbases])

    info = plsc.get_sparse_core_info()
    nw = info.num_cores * info.num_subcores
    reps = -(-n_units // nw)

    mesh = plsc.VectorSubcoreMesh(core_axis_name="c", subcore_axis_name="s")
    sc_params = pltpu.CompilerParams(needs_layout_passes=False, use_tc_tiling_on_sc=False)
    qkc, vc, idxo, cnto = pl.kernel(
        functools.partial(_sc_compact_gather, n_pairs, n_cls, n_ch, b, hw, c,
                          n_units, nw, reps),
        out_type=[
            jax.ShapeDtypeStruct((n_units, hw, c), jnp.float32),
            jax.ShapeDtypeStruct((n_pairs, hw, 8), jnp.float32),
            jax.ShapeDtypeStruct((n_units * nchp1 * CH,), jnp.int32),
            jax.ShapeDtypeStruct((n_units * 16,), jnp.int32),
        ],
        mesh=mesh,
        scratch_types=[
            pltpu.VMEM((hw,), jnp.int32),
            pltpu.VMEM((nchp1 * CH,), jnp.int32),
            pltpu.VMEM((nchp1 * CH,), jnp.int32),
            pltpu.VMEM((CH, c), jnp.float32),
            pltpu.VMEM((CH, 8), jnp.float32),
            pltpu.VMEM((16,), jnp.int32),
            pltpu.VMEM((16,), jnp.int32),
            pltpu.VMEM((16,), jnp.int32),
            pltpu.VMEM((16,), jnp.int32),
            pltpu.SemaphoreType.DMA,
        ],
        compiler_params=sc_params,
    )(tabcat, vT, labcat, bases)

    colc = pl.pallas_call(
        functools.partial(_tc_attn, hw),
        grid_spec=pltpu.PrefetchScalarGridSpec(
            num_scalar_prefetch=1,
            grid=(n_pairs, n_rb),
            in_specs=[
                pl.BlockSpec((1, hw, c), lambda p, r, s: (p, 0, 0)),
                pl.BlockSpec((1, hw, c), lambda p, r, s: (n_pairs + p, 0, 0)),
                pl.BlockSpec((1, hw, 8), lambda p, r, s: (p, 0, 0)),
            ],
            out_specs=pl.BlockSpec((1, RB, 8), lambda p, r, s: (p, r, 0)),
            scratch_shapes=[
                pltpu.VMEM((hw, c), jnp.bfloat16),
                pltpu.VMEM((hw, c), jnp.bfloat16),
                pltpu.VMEM((hw, 8), jnp.bfloat16),
            ],
        ),
        out_shape=jax.ShapeDtypeStruct((n_pairs, hw, 8), jnp.float32),
    )(cnto, qkc, qkc, vc)

    dense = pl.kernel(
        functools.partial(_sc_scatter, n_pairs, hw, nchp1),
        out_type=jax.ShapeDtypeStruct((n_pairs * hwp, 8), jnp.float32),
        mesh=mesh,
        scratch_types=[
            pltpu.VMEM((nchp1 * CH,), jnp.int32),
            pltpu.VMEM((CH, 8), jnp.float32),
            pltpu.VMEM((16,), jnp.int32),
            pltpu.SemaphoreType.DMA,
        ],
        compiler_params=sc_params,
    )(colc, idxo, cnto).reshape(n_pairs, hwp, 8)

    chm = 512
    dense_hw = dense[:, :hw, :]
    canvasT = pl.pallas_call(
        functools.partial(_tc_merge, n_cls, n_ch, chm),
        grid_spec=pltpu.PrefetchScalarGridSpec(
            num_scalar_prefetch=1,
            grid=(b, hw // chm),
            in_specs=[
                pl.BlockSpec((n_cls, chm, 8), lambda i, j, s: (i, j, 0)),
                pl.BlockSpec((1, chm, n_ch), lambda i, j, s: (i, j, 0)),
            ],
            out_specs=pl.BlockSpec((1, chm, 3), lambda i, j, s: (i, j, 0)),
        ),
        out_shape=jax.ShapeDtypeStruct((b, hw, 3), jnp.float32),
    )(cnto, dense_hw, glT)
    return jnp.swapaxes(canvasT, 1, 2).reshape(b, 3, h, w)
